# baseline (speedup 1.0000x reference)
import functools

import jax
import jax.numpy as jnp
from jax.experimental import pallas as pl
from jax.experimental.pallas import tpu as pltpu

f32 = jnp.float32
bf16 = jnp.bfloat16


def _silu(v):
    return v * jax.nn.sigmoid(v)


def _ln(v, eps=1e-5):
    m = jnp.mean(v, axis=-1, keepdims=True)
    c = v - m
    var = jnp.mean(c * c, axis=-1, keepdims=True)
    return c * jax.lax.rsqrt(var + eps)


def _dot_t(a, b):
    return jax.lax.dot_general(a, b, (((1,), (1,)), ((), ())),
                               preferred_element_type=f32)


def _combine_body(A, wo_ref, we_ref, wd_ref, wa_ref, awr_ref, wpre_ref,
                  cc_ref, wpat_ref):
    cc_ref[:, :A] = jnp.dot(wo_ref[...].astype(bf16),
                            we_ref[...].astype(bf16),
                            preferred_element_type=f32).astype(bf16)
    cc_ref[:, A:] = jnp.dot(wd_ref[...].astype(bf16),
                            wa_ref[...].astype(bf16),
                            preferred_element_type=f32).astype(bf16)
    wpat_ref[...] = jnp.dot(awr_ref[...].astype(bf16),
                            wpre_ref[...].astype(bf16),
                            preferred_element_type=f32).astype(bf16)


def _fused_body(E, A, H, BT, x_ref, ew_ref, wup_ref, wgate_ref, wpre_ref,
                wpost_ref, lng_ref, lnb_ref, wpat_ref, ag_ref, ab_ref,
                wdown_ref, cc_ref, out_ref,
                wug_ref, wdb_ref, hid_ref, ain_ref, aout_ref, acc_ref):
    ph = pl.program_id(1)
    j = pl.program_id(2)
    first = jnp.logical_and(pl.program_id(0) == 0,
                            jnp.logical_and(ph == 0, j == 0))

    @pl.when(first)
    def _cast_weights():
        wug_ref[:H, :] = wup_ref[...].astype(bf16)
        wug_ref[H:2 * H, :] = wgate_ref[...].astype(bf16)
        wug_ref[2 * H:2 * H + A, :] = wpre_ref[...].astype(bf16)
        wug_ref[2 * H + A:, :] = wpat_ref[...]
        wdb_ref[...] = wdown_ref[...].astype(bf16)

    rows = pl.ds(j * BT, BT)

    @pl.when(ph == 0)
    def _phase0():
        xb = x_ref[...].astype(bf16)
        ug = _dot_t(xb, wug_ref[...])
        up = ug[:, :H]
        gate = ug[:, H:2 * H]
        p = ug[:, 2 * H:2 * H + A]
        t_all = ug[:, 2 * H + A:]
        hid = _silu(gate) * up
        hid_ref[rows, :] = hid.astype(bf16)

        lng = lng_ref[...]
        lnb = lnb_ref[...]
        ain_ref[rows, :] = (_ln(p) * lng + lnb).astype(bf16)
        ao = _dot_t(hid.astype(bf16), wpost_ref[...])
        aout_ref[rows, :] = (_ln(ao) * lng + lnb).astype(bf16)

        w = ew_ref[...]
        coef = jnp.where(w > 0, w, 0.0)
        acc = jnp.zeros_like(p)
        for i in range(E):
            t = _ln(t_all[:, i * A:(i + 1) * A])
            t = t * ag_ref[i:i + 1, :] + ab_ref[i:i + 1, :]
            acc = acc + coef[:, i:i + 1] * t
        acc_ref[rows, :] = acc.astype(bf16)

    @pl.when(ph == 1)
    def _phase1():
        qb = ain_ref[rows, :]
        scores = _dot_t(qb, aout_ref[...])
        sc = jnp.clip(scores, -5.0, 5.0)
        aw = _silu(sc)
        adapt = jnp.dot(aw.astype(bf16), ain_ref[...],
                        preferred_element_type=f32)
        wsum = jnp.sum(ew_ref[...], axis=1, keepdims=True)
        adaptw = (adapt * wsum).astype(bf16)
        shared = _dot_t(hid_ref[rows, :], wdb_ref[...])
        both = jnp.concatenate([acc_ref[rows, :], adaptw], axis=1)
        proj = _dot_t(both, cc_ref[...])
        out_ref[...] = shared * wsum + 0.1 * proj


def kernel(x, expert_weights, W_up, W_gate, W_down, W_pre, W_post, ln_g, ln_b,
           W_aproj, adapter_W, adapter_g, adapter_b, W_eproj, W_oproj):
    B, S, D = x.shape
    E = expert_weights.shape[-1]
    H = W_up.shape[0]
    A = W_pre.shape[0]
    N = B * S
    BT = 256
    NSB = S // BT

    xt = x.reshape(N, D)
    ew = expert_weights.reshape(N, E)
    lng = ln_g.reshape(1, A).astype(f32)
    lnb = ln_b.reshape(1, A).astype(f32)
    aWr = adapter_W.reshape(E * A, A)

    CC, WpaT = pl.pallas_call(
        functools.partial(_combine_body, A),
        out_shape=[jax.ShapeDtypeStruct((D, 2 * A), bf16),
                   jax.ShapeDtypeStruct((E * A, D), bf16)],
    )(W_oproj, W_eproj, W_down, W_aproj, aWr, W_pre)

    const2 = lambda shape: pl.BlockSpec(shape, lambda b, ph, j: (0, 0))
    out = pl.pallas_call(
        functools.partial(_fused_body, E, A, H, BT),
        grid=(B, 2, NSB),
        in_specs=[
            pl.BlockSpec(
                (BT, D),
                lambda b, ph, j: (b * NSB + jnp.where(ph == 0, j, NSB - 1),
                                  0)),
            pl.BlockSpec((BT, E), lambda b, ph, j: (b * NSB + j, 0)),
            const2((H, D)),
            const2((H, D)),
            const2((A, D)),
            const2((A, H)),
            const2((1, A)),
            const2((1, A)),
            const2((E * A, D)),
            const2((E, A)),
            const2((E, A)),
            const2((D, H)),
            const2((D, 2 * A)),
        ],
        out_specs=pl.BlockSpec(
            (BT, D),
            lambda b, ph, j: (b * NSB + jnp.where(ph == 1, j, 0), 0)),
        out_shape=jax.ShapeDtypeStruct((N, D), f32),
        scratch_shapes=[
            pltpu.VMEM((2 * H + A + E * A, D), bf16),
            pltpu.VMEM((D, H), bf16),
            pltpu.VMEM((S, H), bf16),
            pltpu.VMEM((S, A), bf16),
            pltpu.VMEM((S, A), bf16),
            pltpu.VMEM((S, A), bf16),
        ],
        compiler_params=pltpu.CompilerParams(
            dimension_semantics=("arbitrary", "arbitrary", "arbitrary"),
            vmem_limit_bytes=110 * 1024 * 1024),
    )(xt, ew, W_up, W_gate, W_pre, W_post.astype(bf16),
      lng, lnb, WpaT, adapter_g.astype(f32), adapter_b.astype(f32),
      W_down, CC)

    return out.reshape(B, S, D)


# speedup vs baseline: 1.1138x; 1.1138x over previous
import functools

import jax
import jax.numpy as jnp
from jax.experimental import pallas as pl
from jax.experimental.pallas import tpu as pltpu

f32 = jnp.float32
bf16 = jnp.bfloat16


def _silu(v):
    return v * jax.nn.sigmoid(v)


def _ln(v, eps=1e-5):
    m = jnp.mean(v, axis=-1, keepdims=True)
    c = v - m
    var = jnp.mean(c * c, axis=-1, keepdims=True)
    return c * jax.lax.rsqrt(var + eps)


def _dot_t(a, b):
    return jax.lax.dot_general(a, b, (((1,), (1,)), ((), ())),
                               preferred_element_type=f32)


def _combine_body(A, wo_ref, we_ref, wd_ref, wa_ref, awr_ref, wpre_ref,
                  cc_ref, wpat_ref, wdb_ref):
    wdb_ref[...] = wd_ref[...].astype(bf16)
    cc_ref[:, :A] = jnp.dot(wo_ref[...].astype(bf16),
                            we_ref[...].astype(bf16),
                            preferred_element_type=f32).astype(bf16)
    cc_ref[:, A:] = jnp.dot(wd_ref[...].astype(bf16),
                            wa_ref[...].astype(bf16),
                            preferred_element_type=f32).astype(bf16)
    wpat_ref[...] = jnp.dot(awr_ref[...].astype(bf16),
                            wpre_ref[...].astype(bf16),
                            preferred_element_type=f32).astype(bf16)


def _fused_body(E, A, H, BT, x_ref, ew_ref, wup_ref, wgate_ref, wpre_ref,
                wpost_ref, lng_ref, lnb_ref, wpat_ref, ag_ref, ab_ref,
                wdb_ref, cc_ref, out_ref,
                wug_ref, hid_ref, ain_ref, aout_ref, acc_ref):
    ph = pl.program_id(1)
    j = pl.program_id(2)
    first = jnp.logical_and(pl.program_id(0) == 0,
                            jnp.logical_and(ph == 0, j == 0))

    @pl.when(first)
    def _cast_weights():
        wug_ref[:H, :] = wup_ref[...].astype(bf16)
        wug_ref[H:2 * H, :] = wgate_ref[...].astype(bf16)
        wug_ref[2 * H:2 * H + A, :] = wpre_ref[...].astype(bf16)
        wug_ref[2 * H + A:, :] = wpat_ref[...]

    rows = pl.ds(j * BT, BT)

    @pl.when(ph == 0)
    def _phase0():
        xb = x_ref[...].astype(bf16)
        ug = _dot_t(xb, wug_ref[...])
        up = ug[:, :H]
        gate = ug[:, H:2 * H]
        p = ug[:, 2 * H:2 * H + A]
        t_all = ug[:, 2 * H + A:]
        hid = _silu(gate) * up
        hid_ref[rows, :] = hid.astype(bf16)

        lng = lng_ref[...]
        lnb = lnb_ref[...]
        ain_ref[rows, :] = (_ln(p) * lng + lnb).astype(bf16)
        ao = _dot_t(hid.astype(bf16), wpost_ref[...])
        aout_ref[rows, :] = (_ln(ao) * lng + lnb).astype(bf16)

        w = ew_ref[...]
        coef = jnp.where(w > 0, w, 0.0)
        acc = jnp.zeros_like(p)
        for i in range(E):
            t = _ln(t_all[:, i * A:(i + 1) * A])
            t = t * ag_ref[i:i + 1, :] + ab_ref[i:i + 1, :]
            acc = acc + coef[:, i:i + 1] * t
        acc_ref[rows, :] = acc.astype(bf16)

    @pl.when(ph == 1)
    def _phase1():
        qb = ain_ref[rows, :]
        scores = _dot_t(qb, aout_ref[...])
        sc = jnp.clip(scores, -5.0, 5.0)
        aw = _silu(sc)
        adapt = jnp.dot(aw.astype(bf16), ain_ref[...],
                        preferred_element_type=f32)
        wsum = jnp.sum(ew_ref[...], axis=1, keepdims=True)
        adaptw = (adapt * wsum).astype(bf16)
        shared = _dot_t(hid_ref[rows, :], wdb_ref[...])
        both = jnp.concatenate([acc_ref[rows, :], adaptw], axis=1)
        proj = _dot_t(both, cc_ref[...])
        out_ref[...] = shared * wsum + 0.1 * proj


def kernel(x, expert_weights, W_up, W_gate, W_down, W_pre, W_post, ln_g, ln_b,
           W_aproj, adapter_W, adapter_g, adapter_b, W_eproj, W_oproj):
    B, S, D = x.shape
    E = expert_weights.shape[-1]
    H = W_up.shape[0]
    A = W_pre.shape[0]
    N = B * S
    BT = 512
    NSB = S // BT

    xt = x.reshape(N, D)
    ew = expert_weights.reshape(N, E)
    lng = ln_g.reshape(1, A).astype(f32)
    lnb = ln_b.reshape(1, A).astype(f32)
    aWr = adapter_W.reshape(E * A, A)

    CC, WpaT, Wdb = pl.pallas_call(
        functools.partial(_combine_body, A),
        out_shape=[jax.ShapeDtypeStruct((D, 2 * A), bf16),
                   jax.ShapeDtypeStruct((E * A, D), bf16),
                   jax.ShapeDtypeStruct((D, H), bf16)],
    )(W_oproj, W_eproj, W_down, W_aproj, aWr, W_pre)

    const2 = lambda shape: pl.BlockSpec(shape, lambda b, ph, j: (0, 0))
    out = pl.pallas_call(
        functools.partial(_fused_body, E, A, H, BT),
        grid=(B, 2, NSB),
        in_specs=[
            pl.BlockSpec(
                (BT, D),
                lambda b, ph, j: (b * NSB + jnp.where(ph == 0, j, NSB - 1),
                                  0)),
            pl.BlockSpec((BT, E), lambda b, ph, j: (b * NSB + j, 0)),
            const2((H, D)),
            const2((H, D)),
            const2((A, D)),
            const2((A, H)),
            const2((1, A)),
            const2((1, A)),
            const2((E * A, D)),
            const2((E, A)),
            const2((E, A)),
            const2((D, H)),
            const2((D, 2 * A)),
        ],
        out_specs=pl.BlockSpec(
            (BT, D),
            lambda b, ph, j: (b * NSB + jnp.where(ph == 1, j, 0), 0)),
        out_shape=jax.ShapeDtypeStruct((N, D), f32),
        scratch_shapes=[
            pltpu.VMEM((2 * H + A + E * A, D), bf16),
            pltpu.VMEM((S, H), bf16),
            pltpu.VMEM((S, A), bf16),
            pltpu.VMEM((S, A), bf16),
            pltpu.VMEM((S, A), bf16),
        ],
        compiler_params=pltpu.CompilerParams(
            dimension_semantics=("arbitrary", "arbitrary", "arbitrary"),
            vmem_limit_bytes=110 * 1024 * 1024),
    )(xt, ew, W_up, W_gate, W_pre, W_post.astype(bf16),
      lng, lnb, WpaT, adapter_g.astype(f32), adapter_b.astype(f32),
      Wdb, CC)

    return out.reshape(B, S, D)
